# baseline (speedup 1.0000x reference)
import functools

import jax
import jax.numpy as jnp
from jax.experimental import pallas as pl
from jax.experimental.pallas import tpu as pltpu

_P = 26
_B = 16384
_K = 64
_BB = 512
_NSTEPS = _B // _BB
_R = 8
_ROWS = _P * _R
_N = float(_B * _K)
_LAMBDA = 0.1


def _fused_body(x_ref, pos_ref, out_hbm, loss_ref, obuf, g_ref, s_ref, sem):
    step = pl.program_id(0)
    slot = step % 2
    x = x_ref[...]

    @pl.when(step >= 2)
    def _():
        pltpu.make_async_copy(
            obuf.at[slot], out_hbm.at[:, pl.ds((step - 2) * _BB, _BB), :],
            sem.at[slot]).wait()

    obuf[slot] = x + pos_ref[...]
    pltpu.make_async_copy(
        obuf.at[slot], out_hbm.at[:, pl.ds(step * _BB, _BB), :],
        sem.at[slot]).start()

    half = _BB // 2
    y = jnp.concatenate([x[:, :half, :], x[:, half:, :]], axis=2)
    chunk = half // _R
    y8 = jnp.concatenate(
        [y[:, r * chunk:(r + 1) * chunk, :] for r in range(_R)], axis=0)
    xr = y8.reshape(_ROWS, chunk * 2 * _K)
    s = jnp.sum(xr, axis=1, keepdims=True)
    xb = xr.astype(jnp.bfloat16)
    g = jax.lax.dot_general(xb, xb, (((1,), (1,)), ((), ())),
                            preferred_element_type=jnp.float32)

    @pl.when(step == 0)
    def _():
        g_ref[...] = g
        s_ref[...] = s

    @pl.when(step > 0)
    def _():
        g_ref[...] += g
        s_ref[...] += s

    @pl.when(step == _NSTEPS - 1)
    def _epilogue():
        big_g = g_ref[...]
        big_s = s_ref[...]
        ai = jax.lax.broadcasted_iota(jnp.int32, (_ROWS, _ROWS), 0)
        bi = jax.lax.broadcasted_iota(jnp.int32, (_ROWS, _ROWS), 1)
        keep = (ai // _P) == (bi // _P)
        gm = jnp.where(keep, big_g, 0.0)
        pi = jax.lax.broadcasted_iota(jnp.int32, (_P, _ROWS), 0)
        aj = jax.lax.broadcasted_iota(jnp.int32, (_P, _ROWS), 1)
        fold = ((aj % _P) == pi).astype(jnp.float32)
        t = jax.lax.dot_general(fold, gm, (((1,), (1,)), ((), ())),
                                preferred_element_type=jnp.float32)
        raw26 = jax.lax.dot_general(t, fold, (((1,), (1,)), ((), ())),
                                    preferred_element_type=jnp.float32)
        s26 = jax.lax.dot_general(fold, big_s, (((1,), (0,)), ((), ())),
                                  preferred_element_type=jnp.float32)
        gc = raw26 - (s26 * s26.T) * (1.0 / _N)
        qi = jax.lax.broadcasted_iota(jnp.int32, (_P, _P), 0)
        qj = jax.lax.broadcasted_iota(jnp.int32, (_P, _P), 1)
        eye = (qi == qj).astype(jnp.float32)
        diag = jnp.sum(jnp.where(qi == qj, gc, 0.0), axis=1, keepdims=True)
        nrm = jnp.sqrt(diag)
        denom = (nrm + 1e-8) * (nrm + 1e-8).T
        off = gc / denom - eye
        row_sq = jnp.sum(off * off, axis=1, keepdims=True)
        total = jnp.sum(row_sq, axis=0, keepdims=True)
        loss_ref[...] = total * (_LAMBDA / (_P * (_P - 1)))
        pltpu.make_async_copy(
            obuf.at[(step - 1) % 2],
            out_hbm.at[:, pl.ds((step - 1) * _BB, _BB), :],
            sem.at[(step - 1) % 2]).wait()
        pltpu.make_async_copy(
            obuf.at[slot], out_hbm.at[:, pl.ds(step * _BB, _BB), :],
            sem.at[slot]).wait()


@functools.partial(jax.jit, static_argnames=("interpret",))
def kernel(partition_outputs, pos_table, interpret=False):
    pos3 = pos_table.reshape(_P, 1, _K)

    processed, loss11 = pl.pallas_call(
        _fused_body,
        grid=(_NSTEPS,),
        in_specs=[
            pl.BlockSpec((_P, _BB, _K), lambda i: (0, i, 0)),
            pl.BlockSpec((_P, 1, _K), lambda i: (0, 0, 0)),
        ],
        out_specs=[
            pl.BlockSpec(memory_space=pl.ANY),
            pl.BlockSpec((1, 1), lambda i: (0, 0)),
        ],
        out_shape=[
            jax.ShapeDtypeStruct((_P, _B, _K), jnp.float32),
            jax.ShapeDtypeStruct((1, 1), jnp.float32),
        ],
        scratch_shapes=[
            pltpu.VMEM((2, _P, _BB, _K), jnp.float32),
            pltpu.VMEM((_ROWS, _ROWS), jnp.float32),
            pltpu.VMEM((_ROWS, 1), jnp.float32),
            pltpu.SemaphoreType.DMA((2,)),
        ],
        compiler_params=pltpu.CompilerParams(
            dimension_semantics=("arbitrary",)),
        interpret=interpret,
    )(partition_outputs, pos3)

    return processed, loss11[0, 0]


# speedup vs baseline: 5.6039x; 5.6039x over previous
import functools

import jax
import jax.numpy as jnp
from jax.experimental import pallas as pl
from jax.experimental.pallas import tpu as pltpu

_P = 26
_B = 16384
_K = 64
_BBL = 1024
_NSTEPS = _B // _BBL
_R = 8
_ROWS = _P * _R
_N = float(_B * _K)
_LAMBDA = 0.1


def _fused_body(x_ref, pos_ref, out_ref, loss_ref, g_ref, s_ref):
    step = pl.program_id(0)
    x = x_ref[...]

    out_ref[...] = x + pos_ref[...]

    chunk = _BBL // _R
    y8 = jnp.concatenate(
        [x[:, :, r * chunk:(r + 1) * chunk] for r in range(_R)], axis=0)
    xr = y8.reshape(_ROWS, _K * chunk)
    s = jnp.sum(xr, axis=1, keepdims=True)
    xb = xr.astype(jnp.bfloat16)
    g = jax.lax.dot_general(xb, xb, (((1,), (1,)), ((), ())),
                            preferred_element_type=jnp.float32)

    @pl.when(step == 0)
    def _init():
        g_ref[...] = g
        s_ref[...] = s
        loss_ref[...] = jnp.zeros((1, 1), jnp.float32)

    @pl.when(step > 0)
    def _acc():
        g_ref[...] += g
        s_ref[...] += s

    @pl.when(step == _NSTEPS - 1)
    def _epilogue():
        big_g = g_ref[...]
        big_s = s_ref[...]
        ai = jax.lax.broadcasted_iota(jnp.int32, (_ROWS, _ROWS), 0)
        bi = jax.lax.broadcasted_iota(jnp.int32, (_ROWS, _ROWS), 1)
        keep = (ai // _P) == (bi // _P)
        gm = jnp.where(keep, big_g, 0.0)
        pi = jax.lax.broadcasted_iota(jnp.int32, (_P, _ROWS), 0)
        aj = jax.lax.broadcasted_iota(jnp.int32, (_P, _ROWS), 1)
        fold = ((aj % _P) == pi).astype(jnp.float32)
        t = jax.lax.dot_general(fold, gm, (((1,), (1,)), ((), ())),
                                preferred_element_type=jnp.float32)
        raw26 = jax.lax.dot_general(t, fold, (((1,), (1,)), ((), ())),
                                    preferred_element_type=jnp.float32)
        s26 = jax.lax.dot_general(fold, big_s, (((1,), (0,)), ((), ())),
                                  preferred_element_type=jnp.float32)
        gc = raw26 - (s26 * s26.T) * (1.0 / _N)
        qi = jax.lax.broadcasted_iota(jnp.int32, (_P, _P), 0)
        qj = jax.lax.broadcasted_iota(jnp.int32, (_P, _P), 1)
        eye = (qi == qj).astype(jnp.float32)
        diag = jnp.sum(jnp.where(qi == qj, gc, 0.0), axis=1, keepdims=True)
        nrm = jnp.sqrt(diag)
        denom = (nrm + 1e-8) * (nrm + 1e-8).T
        off = gc / denom - eye
        row_sq = jnp.sum(off * off, axis=1, keepdims=True)
        total = jnp.sum(row_sq, axis=0, keepdims=True)
        loss_ref[...] = total * (_LAMBDA / (_P * (_P - 1)))


@functools.partial(jax.jit, static_argnames=("interpret",))
def kernel(partition_outputs, pos_table, interpret=False):
    x_t = jnp.transpose(partition_outputs, (0, 2, 1))
    pos3 = pos_table.reshape(_P, _K, 1)

    out_t, loss11 = pl.pallas_call(
        _fused_body,
        grid=(_NSTEPS,),
        in_specs=[
            pl.BlockSpec((_P, _K, _BBL), lambda i: (0, 0, i)),
            pl.BlockSpec((_P, _K, 1), lambda i: (0, 0, 0)),
        ],
        out_specs=[
            pl.BlockSpec((_P, _K, _BBL), lambda i: (0, 0, i)),
            pl.BlockSpec((1, 1), lambda i: (0, 0)),
        ],
        out_shape=[
            jax.ShapeDtypeStruct((_P, _K, _B), jnp.float32),
            jax.ShapeDtypeStruct((1, 1), jnp.float32),
        ],
        scratch_shapes=[
            pltpu.VMEM((_ROWS, _ROWS), jnp.float32),
            pltpu.VMEM((_ROWS, 1), jnp.float32),
        ],
        compiler_params=pltpu.CompilerParams(
            dimension_semantics=("arbitrary",)),
        interpret=interpret,
    )(x_t, pos3)

    processed = jnp.transpose(out_t, (0, 2, 1))
    return processed, loss11[0, 0]
